# baseline (speedup 1.0000x reference)
import functools

import jax
import jax.numpy as jnp
from jax import lax
from jax.experimental import pallas as pl
from jax.experimental.pallas import tpu as pltpu
from jax.experimental.pallas import tpu_sc as plsc


def _proj_body(emb_ref, w_ref, b_ref, out_ref):
    out_ref[...] = lax.dot_general(
        emb_ref[...], w_ref[...],
        dimension_numbers=(((1,), (1,)), ((), ())),
        preferred_element_type=jnp.float32,
    ) + b_ref[...]


def _project_table(emb_table, W1, b1):
    V, D = emb_table.shape
    return pl.pallas_call(
        _proj_body,
        out_shape=jax.ShapeDtypeStruct((V, D), jnp.float32),
    )(emb_table, W1, b1.reshape(1, D))


_P = 128
_CHUNK = 128
_NBUF = 4
_PROJ_ROWS = 4096


def _make_sc_gather(num_rows):
    info = plsc.get_sparse_core_info()
    NC, NS = info.num_cores, info.num_subcores
    NW = NC * NS
    rows_per_w = num_rows // NW
    chunks_per_w = rows_per_w // _CHUNK
    quads = chunks_per_w // _NBUF
    mesh = plsc.VectorSubcoreMesh(core_axis_name="c", subcore_axis_name="s")

    @functools.partial(
        pl.kernel,
        mesh=mesh,
        out_type=jax.ShapeDtypeStruct((num_rows, _P), jnp.float32),
        scratch_types=[
            pltpu.VMEM((chunks_per_w, _CHUNK), jnp.int32),
        ] + [pltpu.VMEM((_CHUNK, _P), jnp.float32)] * _NBUF
          + [pltpu.SemaphoreType.DMA] * (2 * _NBUF),
    )
    def sc_kernel(tab_hbm, idx_hbm, out_hbm, idx_v, *bufs_sems):
        bufs = bufs_sems[:_NBUF]
        gsems = bufs_sems[_NBUF:2 * _NBUF]
        osems = bufs_sems[2 * _NBUF:]
        scid = lax.axis_index("c")
        sid = lax.axis_index("s")
        wid = scid * NS + sid
        r0 = wid * rows_per_w

        pltpu.sync_copy(idx_hbm.at[wid], idx_v)

        def fire_gather(j, k):
            return pltpu.async_copy(tab_hbm.at[idx_v.at[j]], bufs[k],
                                    gsems[k])

        def out_slice(j):
            return out_hbm.at[pl.ds(pl.multiple_of(r0 + j * _CHUNK, _CHUNK),
                                    _CHUNK)]

        for k in range(_NBUF):
            fire_gather(k, k)

        def quad(g, last):
            for k in range(_NBUF):
                j = g * _NBUF + k
                pltpu.make_async_copy(tab_hbm.at[idx_v.at[j]], bufs[k],
                                      gsems[k]).wait()
                pltpu.async_copy(bufs[k], out_slice(j), osems[k])
            for k in range(_NBUF):
                j = g * _NBUF + k
                pltpu.make_async_copy(bufs[k], out_slice(j), osems[k]).wait()
                if not last:
                    fire_gather(j + _NBUF, k)

        lax.fori_loop(0, quads - 1, lambda g, c: (quad(g, False), c)[1], 0)
        quad(quads - 1, True)

    return sc_kernel


_RB = 512
_NSLICE = 4


def _relayout_body(in_ref, out_ref):
    out_ref[...] = in_ref[...].reshape(out_ref.shape)


def _relayout_acc_body(in_ref, acc_ref, out_ref):
    del acc_ref
    out_ref[...] = in_ref[...].reshape(out_ref.shape)


def _relayout_slice(out_flat_s, acc, s, num_tokens, D2):
    toks = out_flat_s.shape[0] * _P // D2
    blocks = toks // _RB
    ppr = D2 // _P
    out_shape = jax.ShapeDtypeStruct((num_tokens, D2), jnp.float32)
    if acc is None:
        return pl.pallas_call(
            _relayout_body,
            grid=(blocks,),
            in_specs=[pl.BlockSpec((_RB * ppr, _P), lambda i: (i, 0))],
            out_specs=pl.BlockSpec((_RB, D2),
                                   lambda i, s=s, b=blocks: (i + s * b, 0)),
            out_shape=out_shape,
        )(out_flat_s)
    return pl.pallas_call(
        _relayout_acc_body,
        grid=(blocks,),
        in_specs=[pl.BlockSpec((_RB * ppr, _P), lambda i: (i, 0)),
                  pl.BlockSpec(memory_space=pl.ANY)],
        out_specs=pl.BlockSpec((_RB, D2),
                               lambda i, s=s, b=blocks: (i + s * b, 0)),
        out_shape=out_shape,
        input_output_aliases={1: 0},
    )(out_flat_s, acc)


def kernel(caption_tknID, img_fc, emb_table, W1, b1):
    B, L = caption_tknID.shape
    D = img_fc.shape[1]
    num_tokens = B * L
    pieces = D // _P
    num_rows = num_tokens * 2 * pieces

    proj = _project_table(emb_table, W1, b1)

    tab = jnp.concatenate([
        proj.reshape(emb_table.shape[0] * pieces, _P),
        jnp.zeros((_PROJ_ROWS - emb_table.shape[0] * pieces, _P), jnp.float32),
        img_fc.reshape(B * pieces, _P),
    ], axis=0)

    info = plsc.get_sparse_core_info()
    nw = info.num_cores * info.num_subcores
    tok = caption_tknID.astype(jnp.int32).T.reshape(num_tokens)
    row = jax.lax.broadcasted_iota(jnp.int32, (num_tokens,), 0) % B
    kk = jnp.arange(2 * pieces, dtype=jnp.int32)
    idx8 = jnp.where(
        kk[None, :] < pieces,
        tok[:, None] * pieces + kk[None, :],
        _PROJ_ROWS + row[:, None] * pieces + (kk[None, :] - pieces))
    rows_slice = num_rows // _NSLICE
    idx4d = idx8.reshape(_NSLICE, nw, rows_slice // (nw * _CHUNK), _CHUNK)

    gather_slice = _make_sc_gather(rows_slice)
    acc = None
    for s in range(_NSLICE):
        out_flat_s = gather_slice(tab, idx4d[s])
        acc = _relayout_slice(out_flat_s, acc, s, num_tokens, 2 * D)
    return acc.reshape(L, B, 2 * D).transpose(1, 0, 2)


# speedup vs baseline: 1.5686x; 1.5686x over previous
import functools

import jax
import jax.numpy as jnp
from jax import lax
from jax.experimental import pallas as pl
from jax.experimental.pallas import tpu as pltpu
from jax.experimental.pallas import tpu_sc as plsc


def _proj_body(emb_ref, w_ref, b_ref, out_ref):
    out_ref[...] = lax.dot_general(
        emb_ref[...], w_ref[...],
        dimension_numbers=(((1,), (1,)), ((), ())),
        preferred_element_type=jnp.float32,
    ) + b_ref[...]


def _project_table(emb_table, W1, b1):
    V, D = emb_table.shape
    return pl.pallas_call(
        _proj_body,
        out_shape=jax.ShapeDtypeStruct((V, D), jnp.float32),
    )(emb_table, W1, b1.reshape(1, D))


_P = 128
_CHUNK = 128
_NBUF = 4
_PROJ_ROWS = 4096


def _make_sc_gather(num_rows):
    info = plsc.get_sparse_core_info()
    NC, NS = info.num_cores, info.num_subcores
    NW = NC * NS
    rows_per_w = num_rows // NW
    chunks_per_w = rows_per_w // _CHUNK
    quads = chunks_per_w // _NBUF
    mesh = plsc.VectorSubcoreMesh(core_axis_name="c", subcore_axis_name="s")

    @functools.partial(
        pl.kernel,
        mesh=mesh,
        out_type=jax.ShapeDtypeStruct((num_rows, _P), jnp.float32),
        scratch_types=[
            pltpu.VMEM((chunks_per_w, _CHUNK), jnp.int32),
        ] + [pltpu.VMEM((_CHUNK, _P), jnp.float32)] * _NBUF
          + [pltpu.SemaphoreType.DMA] * (2 * _NBUF),
    )
    def sc_kernel(tab_hbm, idx_hbm, out_hbm, idx_v, *bufs_sems):
        bufs = bufs_sems[:_NBUF]
        gsems = bufs_sems[_NBUF:2 * _NBUF]
        osems = bufs_sems[2 * _NBUF:]
        scid = lax.axis_index("c")
        sid = lax.axis_index("s")
        wid = scid * NS + sid
        r0 = wid * rows_per_w

        pltpu.sync_copy(idx_hbm.at[wid], idx_v)

        def fire_gather(j, k):
            return pltpu.async_copy(tab_hbm.at[idx_v.at[j]], bufs[k],
                                    gsems[k])

        def out_slice(j):
            return out_hbm.at[pl.ds(pl.multiple_of(r0 + j * _CHUNK, _CHUNK),
                                    _CHUNK)]

        for k in range(_NBUF):
            fire_gather(k, k)

        def quad(g, last):
            for k in range(_NBUF):
                j = g * _NBUF + k
                pltpu.make_async_copy(tab_hbm.at[idx_v.at[j]], bufs[k],
                                      gsems[k]).wait()
                pltpu.async_copy(bufs[k], out_slice(j), osems[k])
            for k in range(_NBUF):
                j = g * _NBUF + k
                pltpu.make_async_copy(bufs[k], out_slice(j), osems[k]).wait()
                if not last:
                    fire_gather(j + _NBUF, k)

        lax.fori_loop(0, quads - 1, lambda g, c: (quad(g, False), c)[1], 0)
        quad(quads - 1, True)

    return sc_kernel


_RB = 512
_NSLICE = 4


def _relayout_body(in_ref, out_ref):
    out_ref[...] = in_ref[...].reshape(out_ref.shape)


def _relayout_acc_body(in_ref, acc_ref, out_ref):
    del acc_ref
    out_ref[...] = in_ref[...].reshape(out_ref.shape)


def _relayout_slice(out_flat_s, acc, s, num_tokens, D2):
    toks = out_flat_s.shape[0] * _P // D2
    blocks = toks // _RB
    ppr = D2 // _P
    out_shape = jax.ShapeDtypeStruct((num_tokens, D2), jnp.float32)
    if acc is None:
        return pl.pallas_call(
            _relayout_body,
            grid=(blocks,),
            in_specs=[pl.BlockSpec((_RB * ppr, _P), lambda i: (i, 0))],
            out_specs=pl.BlockSpec((_RB, D2),
                                   lambda i, s=s, b=blocks: (i + s * b, 0)),
            out_shape=out_shape,
        )(out_flat_s)
    return pl.pallas_call(
        _relayout_acc_body,
        grid=(blocks,),
        in_specs=[pl.BlockSpec((_RB * ppr, _P), lambda i: (i, 0)),
                  pl.BlockSpec(memory_space=pl.ANY)],
        out_specs=pl.BlockSpec((_RB, D2),
                               lambda i, s=s, b=blocks: (i + s * b, 0)),
        out_shape=out_shape,
        input_output_aliases={1: 0},
    )(out_flat_s, acc)


def kernel(caption_tknID, img_fc, emb_table, W1, b1):
    B, L = caption_tknID.shape
    D = img_fc.shape[1]
    num_tokens = B * L
    pieces = D // _P
    num_rows = num_tokens * 2 * pieces

    proj = _project_table(emb_table, W1, b1)

    tab = jnp.concatenate([
        proj.reshape(emb_table.shape[0] * pieces, _P),
        jnp.zeros((_PROJ_ROWS - emb_table.shape[0] * pieces, _P), jnp.float32),
        img_fc.reshape(B * pieces, _P),
    ], axis=0)

    info = plsc.get_sparse_core_info()
    nw = info.num_cores * info.num_subcores
    tokT = caption_tknID.astype(jnp.int32).T.reshape(L, B // 8, 1, 8)
    cc = jnp.arange(pieces, dtype=jnp.int32).reshape(1, 1, pieces, 1)
    bb = jax.lax.broadcasted_iota(jnp.int32, (1, B // 8, 1, 8), 1) * 8 \
        + jax.lax.broadcasted_iota(jnp.int32, (1, B // 8, 1, 8), 3)
    word_idx = jnp.broadcast_to(tokT * pieces + cc, (L, B // 8, pieces, 8))
    img_idx = jnp.broadcast_to(_PROJ_ROWS + bb * pieces + cc,
                               (L, B // 8, pieces, 8))
    idx_full = jnp.concatenate([word_idx, img_idx], axis=2)
    idx3d = idx_full.reshape(nw, num_rows // (nw * _CHUNK), _CHUNK)

    out_flat = _make_sc_gather(num_rows)(tab, idx3d)
    x = out_flat.reshape(L, B // 8, 2 * pieces, 8, _P)
    return x.transpose(1, 3, 0, 2, 4).reshape(B, L, 2 * D)
